# baseline (device time: 184068 ns/iter reference)
import jax
import jax.numpy as jnp
from jax import lax
from jax.experimental import pallas as pl
from jax.experimental.pallas import tpu as pltpu

N_DEV = 8


def kernel(x, w_mat):
    m_per, k = x.shape
    _, n_per = w_mat.shape

    def body(x_ref, w_ref, out_ref, comm_ref, send_sems, recv_sems):
        my = lax.axis_index("i")
        left = (my - 1) % N_DEV
        right = (my + 1) % N_DEV

        barrier_sem = pltpu.get_barrier_semaphore()
        for nbr in (left, right):
            pl.semaphore_signal(
                barrier_sem, inc=1,
                device_id=(nbr,), device_id_type=pl.DeviceIdType.MESH,
            )
        pl.semaphore_wait(barrier_sem, 2)

        def gemm_silu_store(chunk, origin):
            y = jnp.dot(chunk, w_ref[...], preferred_element_type=jnp.float32)
            out_ref[pl.ds(origin * m_per, m_per), :] = y * jax.nn.sigmoid(y)

        comm_ref[0] = x_ref[...]
        gemm_silu_store(x_ref[...], my)

        for h in range(N_DEV - 1):
            rdma = pltpu.make_async_remote_copy(
                src_ref=comm_ref.at[h],
                dst_ref=comm_ref.at[h + 1],
                send_sem=send_sems.at[h],
                recv_sem=recv_sems.at[h + 1],
                device_id=(right,),
                device_id_type=pl.DeviceIdType.MESH,
            )
            rdma.start()
            rdma.wait()
            gemm_silu_store(comm_ref[h + 1], (my - (h + 1)) % N_DEV)

    out_shape = jax.ShapeDtypeStruct((N_DEV * m_per, n_per), jnp.float32)
    return pl.pallas_call(
        body,
        out_shape=out_shape,
        in_specs=[
            pl.BlockSpec(memory_space=pltpu.VMEM),
            pl.BlockSpec(memory_space=pltpu.VMEM),
        ],
        out_specs=pl.BlockSpec(memory_space=pltpu.VMEM),
        scratch_shapes=[
            pltpu.VMEM((N_DEV, m_per, k), jnp.float32),
            pltpu.SemaphoreType.DMA((N_DEV,)),
            pltpu.SemaphoreType.DMA((N_DEV,)),
        ],
        compiler_params=pltpu.CompilerParams(collective_id=0),
    )(x, w_mat)


# device time: 106221 ns/iter; 1.7329x vs baseline; 1.7329x over previous
import jax
import jax.numpy as jnp
from jax import lax
from jax.experimental import pallas as pl
from jax.experimental.pallas import tpu as pltpu

N_DEV = 8
F_HOPS = 4
B_HOPS = 3


def kernel(x, w_mat):
    m_per, k = x.shape
    _, n_per = w_mat.shape

    def body(x_ref, w_ref, out_ref, fwd_ref, bwd_ref,
             fsend, frecv, bsend, brecv):
        my = lax.axis_index("i")

        def id_at(pos):
            pos = pos % N_DEV
            return jnp.where(pos < 4, pos, 11 - pos)

        p = jnp.where(my < 4, my, 11 - my)
        right = id_at(p + 1)
        left = id_at(p - 1)

        barrier_sem = pltpu.get_barrier_semaphore()
        for nbr in (left, right):
            pl.semaphore_signal(
                barrier_sem, inc=1,
                device_id=(nbr,), device_id_type=pl.DeviceIdType.MESH,
            )
        pl.semaphore_wait(barrier_sem, 2)

        def gemm_silu_store(chunk, origin):
            y = jnp.dot(chunk, w_ref[...], preferred_element_type=jnp.float32)
            out_ref[pl.ds(origin * m_per, m_per), :] = y * jax.nn.sigmoid(y)

        def fwd_rdma(h):
            return pltpu.make_async_remote_copy(
                src_ref=fwd_ref.at[h], dst_ref=fwd_ref.at[h + 1],
                send_sem=fsend.at[h], recv_sem=frecv.at[h + 1],
                device_id=(right,), device_id_type=pl.DeviceIdType.MESH,
            )

        def bwd_rdma(h):
            return pltpu.make_async_remote_copy(
                src_ref=bwd_ref.at[h], dst_ref=bwd_ref.at[h + 1],
                send_sem=bsend.at[h], recv_sem=brecv.at[h + 1],
                device_id=(left,), device_id_type=pl.DeviceIdType.MESH,
            )

        fwd_ref[0] = x_ref[...]
        bwd_ref[0] = x_ref[...]
        df = [fwd_rdma(h) for h in range(F_HOPS)]
        db = [bwd_rdma(h) for h in range(B_HOPS)]
        df[0].start()
        db[0].start()

        gemm_silu_store(x_ref[...], my)

        for h in range(F_HOPS):
            df[h].wait_recv()
            if h + 1 < F_HOPS:
                df[h + 1].start()
            if h < B_HOPS:
                db[h].wait_recv()
                if h + 1 < B_HOPS:
                    db[h + 1].start()
            gemm_silu_store(fwd_ref[h + 1], id_at(p - (h + 1)))
            if h < B_HOPS:
                gemm_silu_store(bwd_ref[h + 1], id_at(p + (h + 1)))

        for d in df:
            d.wait_send()
        for d in db:
            d.wait_send()

    out_shape = jax.ShapeDtypeStruct((N_DEV * m_per, n_per), jnp.float32)
    return pl.pallas_call(
        body,
        out_shape=out_shape,
        in_specs=[
            pl.BlockSpec(memory_space=pltpu.VMEM),
            pl.BlockSpec(memory_space=pltpu.VMEM),
        ],
        out_specs=pl.BlockSpec(memory_space=pltpu.VMEM),
        scratch_shapes=[
            pltpu.VMEM((F_HOPS + 1, m_per, k), jnp.float32),
            pltpu.VMEM((B_HOPS + 1, m_per, k), jnp.float32),
            pltpu.SemaphoreType.DMA((F_HOPS,)),
            pltpu.SemaphoreType.DMA((F_HOPS + 1,)),
            pltpu.SemaphoreType.DMA((B_HOPS,)),
            pltpu.SemaphoreType.DMA((B_HOPS + 1,)),
        ],
        compiler_params=pltpu.CompilerParams(collective_id=0),
    )(x, w_mat)


# device time: 95069 ns/iter; 1.9362x vs baseline; 1.1173x over previous
import jax
import jax.numpy as jnp
from jax import lax
from jax.experimental import pallas as pl
from jax.experimental.pallas import tpu as pltpu

N_DEV = 8
HOPS = 4


def kernel(x, w_mat):
    m_per, k = x.shape
    _, n_per = w_mat.shape
    half = m_per // 2

    def body(x_ref, w_ref, out_ref, fwd_ref, bwd_ref, fhalf_ref, bhalf_ref,
             fsend, frecv, bsend, brecv):
        my = lax.axis_index("i")

        def id_at(pos):
            pos = pos % N_DEV
            return jnp.where(pos < 4, pos, 11 - pos)

        p = jnp.where(my < 4, my, 11 - my)
        right = id_at(p + 1)
        left = id_at(p - 1)

        barrier_sem = pltpu.get_barrier_semaphore()
        for nbr in (left, right):
            pl.semaphore_signal(
                barrier_sem, inc=1,
                device_id=(nbr,), device_id_type=pl.DeviceIdType.MESH,
            )
        pl.semaphore_wait(barrier_sem, 2)

        def gemm_silu_store(chunk, origin, row0=0):
            y = jnp.dot(chunk, w_ref[...], preferred_element_type=jnp.float32)
            out_ref[pl.ds(origin * m_per + row0, chunk.shape[0]), :] = (
                y * jax.nn.sigmoid(y)
            )

        def fwd_rdma(h):
            if h < HOPS - 1:
                src, dst = fwd_ref.at[h], fwd_ref.at[h + 1]
            else:
                src, dst = fwd_ref.at[h, pl.ds(0, half)], fhalf_ref
            return pltpu.make_async_remote_copy(
                src_ref=src, dst_ref=dst,
                send_sem=fsend.at[h], recv_sem=frecv.at[h],
                device_id=(right,), device_id_type=pl.DeviceIdType.MESH,
            )

        def bwd_rdma(h):
            if h < HOPS - 1:
                src, dst = bwd_ref.at[h], bwd_ref.at[h + 1]
            else:
                src, dst = bwd_ref.at[h, pl.ds(half, half)], bhalf_ref
            return pltpu.make_async_remote_copy(
                src_ref=src, dst_ref=dst,
                send_sem=bsend.at[h], recv_sem=brecv.at[h],
                device_id=(left,), device_id_type=pl.DeviceIdType.MESH,
            )

        fwd_ref[0] = x_ref[...]
        bwd_ref[0] = x_ref[...]
        df = [fwd_rdma(h) for h in range(HOPS)]
        db = [bwd_rdma(h) for h in range(HOPS)]
        df[0].start()
        db[0].start()

        gemm_silu_store(x_ref[...], my)

        antipode = id_at(p + 4)
        for h in range(HOPS):
            df[h].wait_recv()
            if h + 1 < HOPS:
                df[h + 1].start()
            db[h].wait_recv()
            if h + 1 < HOPS:
                db[h + 1].start()
            if h < HOPS - 1:
                gemm_silu_store(fwd_ref[h + 1], id_at(p - (h + 1)))
                gemm_silu_store(bwd_ref[h + 1], id_at(p + (h + 1)))
            else:
                gemm_silu_store(fhalf_ref[...], antipode, row0=0)
                gemm_silu_store(bhalf_ref[...], antipode, row0=half)

        for d in df + db:
            d.wait_send()

    out_shape = jax.ShapeDtypeStruct((N_DEV * m_per, n_per), jnp.float32)
    return pl.pallas_call(
        body,
        out_shape=out_shape,
        in_specs=[
            pl.BlockSpec(memory_space=pltpu.VMEM),
            pl.BlockSpec(memory_space=pltpu.VMEM),
        ],
        out_specs=pl.BlockSpec(memory_space=pltpu.VMEM),
        scratch_shapes=[
            pltpu.VMEM((HOPS, m_per, k), jnp.float32),
            pltpu.VMEM((HOPS, m_per, k), jnp.float32),
            pltpu.VMEM((half, k), jnp.float32),
            pltpu.VMEM((half, k), jnp.float32),
            pltpu.SemaphoreType.DMA((HOPS,)),
            pltpu.SemaphoreType.DMA((HOPS,)),
            pltpu.SemaphoreType.DMA((HOPS,)),
            pltpu.SemaphoreType.DMA((HOPS,)),
        ],
        compiler_params=pltpu.CompilerParams(collective_id=0),
    )(x, w_mat)


# device time: 82443 ns/iter; 2.2327x vs baseline; 1.1531x over previous
import jax
import jax.numpy as jnp
from jax import lax
from jax.experimental import pallas as pl
from jax.experimental.pallas import tpu as pltpu

N_DEV = 8
HOPS = 3


def kernel(x, w_mat):
    m_per, k = x.shape
    _, n_per = w_mat.shape

    def body(x_ref, w_ref, out_ref, fwd_ref, bwd_ref, ant_ref,
             fsend, frecv, bsend, brecv, csend, crecv):
        my = lax.axis_index("i")

        def id_at(pos):
            pos = pos % N_DEV
            return jnp.where(pos < 4, pos, 11 - pos)

        p = jnp.where(my < 4, my, 11 - my)
        right = id_at(p + 1)
        left = id_at(p - 1)
        is_even = (p % 2) == 0
        partner = id_at(jnp.where(is_even, p + 3, p - 3))

        barrier_sem = pltpu.get_barrier_semaphore()
        for nbr in (left, right, partner):
            pl.semaphore_signal(
                barrier_sem, inc=1,
                device_id=(nbr,), device_id_type=pl.DeviceIdType.MESH,
            )
        pl.semaphore_wait(barrier_sem, 3)

        def gemm_silu_store(chunk, origin):
            y = jnp.dot(chunk, w_ref[...], preferred_element_type=jnp.float32)
            out_ref[pl.ds(origin * m_per, m_per), :] = y * jax.nn.sigmoid(y)

        def fwd_rdma(h):
            return pltpu.make_async_remote_copy(
                src_ref=fwd_ref.at[h], dst_ref=fwd_ref.at[h + 1],
                send_sem=fsend.at[h], recv_sem=frecv.at[h],
                device_id=(right,), device_id_type=pl.DeviceIdType.MESH,
            )

        def bwd_rdma(h):
            return pltpu.make_async_remote_copy(
                src_ref=bwd_ref.at[h], dst_ref=bwd_ref.at[h + 1],
                send_sem=bsend.at[h], recv_sem=brecv.at[h],
                device_id=(left,), device_id_type=pl.DeviceIdType.MESH,
            )

        def chord_rdma(src):
            return pltpu.make_async_remote_copy(
                src_ref=src, dst_ref=ant_ref,
                send_sem=csend.at[0], recv_sem=crecv.at[0],
                device_id=(partner,), device_id_type=pl.DeviceIdType.MESH,
            )

        fwd_ref[0] = x_ref[...]
        bwd_ref[0] = x_ref[...]
        df = [fwd_rdma(h) for h in range(HOPS)]
        db = [bwd_rdma(h) for h in range(HOPS)]
        chord_even = chord_rdma(fwd_ref.at[1])
        chord_odd = chord_rdma(bwd_ref.at[1])
        df[0].start()
        db[0].start()

        gemm_silu_store(x_ref[...], my)

        df[0].wait_recv()
        df[1].start()
        db[0].wait_recv()
        db[1].start()

        @pl.when(is_even)
        def _():
            chord_even.start()

        @pl.when(jnp.logical_not(is_even))
        def _():
            chord_odd.start()

        gemm_silu_store(fwd_ref[1], id_at(p - 1))
        gemm_silu_store(bwd_ref[1], id_at(p + 1))

        df[1].wait_recv()
        df[2].start()
        db[1].wait_recv()
        db[2].start()
        gemm_silu_store(fwd_ref[2], id_at(p - 2))
        gemm_silu_store(bwd_ref[2], id_at(p + 2))
        chord_even.wait_recv()
        gemm_silu_store(ant_ref[...], id_at(p + 4))

        df[2].wait_recv()
        gemm_silu_store(fwd_ref[3], id_at(p - 3))
        db[2].wait_recv()
        gemm_silu_store(bwd_ref[3], id_at(p + 3))

        for d in df + db:
            d.wait_send()
        chord_even.wait_send()

    out_shape = jax.ShapeDtypeStruct((N_DEV * m_per, n_per), jnp.float32)
    return pl.pallas_call(
        body,
        out_shape=out_shape,
        in_specs=[
            pl.BlockSpec(memory_space=pltpu.VMEM),
            pl.BlockSpec(memory_space=pltpu.VMEM),
        ],
        out_specs=pl.BlockSpec(memory_space=pltpu.VMEM),
        scratch_shapes=[
            pltpu.VMEM((HOPS + 1, m_per, k), jnp.float32),
            pltpu.VMEM((HOPS + 1, m_per, k), jnp.float32),
            pltpu.VMEM((m_per, k), jnp.float32),
            pltpu.SemaphoreType.DMA((HOPS,)),
            pltpu.SemaphoreType.DMA((HOPS,)),
            pltpu.SemaphoreType.DMA((HOPS,)),
            pltpu.SemaphoreType.DMA((HOPS,)),
            pltpu.SemaphoreType.DMA((1,)),
            pltpu.SemaphoreType.DMA((1,)),
        ],
        compiler_params=pltpu.CompilerParams(collective_id=0),
    )(x, w_mat)


# device time: 70814 ns/iter; 2.5993x vs baseline; 1.1642x over previous
import jax
import jax.numpy as jnp
from jax import lax
from jax.experimental import pallas as pl
from jax.experimental.pallas import tpu as pltpu

N_DEV = 8


def kernel(x, w_mat):
    m_per, k = x.shape
    _, n_per = w_mat.shape
    half = m_per // 2

    def body(x_ref, w_ref, out_ref, fwd_ref, bwd_ref, ant_ref, chd_ref,
             ftop_ref, fbot_ref, fsend, frecv, bsend, brecv, csend, crecv):
        my = lax.axis_index("i")

        def id_at(pos):
            pos = pos % N_DEV
            return jnp.where(pos < 4, pos, 11 - pos)

        p = jnp.where(my < 4, my, 11 - my)
        right = id_at(p + 1)
        left = id_at(p - 1)
        is_even = (p % 2) == 0
        partner = id_at(jnp.where(is_even, p + 3, p - 3))

        barrier_sem = pltpu.get_barrier_semaphore()
        for nbr in (left, right, partner):
            pl.semaphore_signal(
                barrier_sem, inc=1,
                device_id=(nbr,), device_id_type=pl.DeviceIdType.MESH,
            )
        pl.semaphore_wait(barrier_sem, 3)

        def gemm_silu_store(chunk, origin, row0=0):
            y = jnp.dot(chunk, w_ref[...], preferred_element_type=jnp.float32)
            out_ref[pl.ds(origin * m_per + row0, chunk.shape[0]), :] = (
                y * jax.nn.sigmoid(y)
            )

        def rdma(src, dst, ssem, rsem, dev):
            return pltpu.make_async_remote_copy(
                src_ref=src, dst_ref=dst, send_sem=ssem, recv_sem=rsem,
                device_id=(dev,), device_id_type=pl.DeviceIdType.MESH,
            )

        f0 = rdma(fwd_ref.at[0], fwd_ref.at[1], fsend.at[0], frecv.at[0], right)
        f1 = rdma(fwd_ref.at[1], fwd_ref.at[2], fsend.at[1], frecv.at[1], right)
        f2_odd = rdma(fwd_ref.at[2, pl.ds(0, half)], ftop_ref,
                      fsend.at[2], frecv.at[2], right)
        f2_even = rdma(ant_ref.at[pl.ds(0, half)], ftop_ref,
                       fsend.at[2], frecv.at[2], right)
        b0 = rdma(bwd_ref.at[0], bwd_ref.at[1], bsend.at[0], brecv.at[0], left)
        b1 = rdma(bwd_ref.at[1], bwd_ref.at[2], bsend.at[1], brecv.at[1], left)
        b2_odd = rdma(ant_ref.at[pl.ds(half, half)], fbot_ref,
                      bsend.at[2], brecv.at[2], left)
        b2_even = rdma(bwd_ref.at[2, pl.ds(half, half)], fbot_ref,
                       bsend.at[2], brecv.at[2], left)
        c_own = rdma(fwd_ref.at[0], chd_ref, csend.at[0], crecv.at[0], partner)
        c_rel_even = rdma(fwd_ref.at[1], ant_ref, csend.at[1], crecv.at[1],
                          partner)
        c_rel_odd = rdma(bwd_ref.at[1], ant_ref, csend.at[1], crecv.at[1],
                         partner)

        fwd_ref[0] = x_ref[...]
        bwd_ref[0] = x_ref[...]
        f0.start()
        b0.start()
        c_own.start()

        gemm_silu_store(x_ref[...], my)

        f0.wait_recv()
        f1.start()
        b0.wait_recv()
        b1.start()

        @pl.when(is_even)
        def _():
            c_rel_even.start()

        @pl.when(jnp.logical_not(is_even))
        def _():
            c_rel_odd.start()

        c_own.wait_recv()
        gemm_silu_store(chd_ref[...], id_at(jnp.where(is_even, p + 3, p - 3)))
        gemm_silu_store(fwd_ref[1], id_at(p - 1))
        gemm_silu_store(bwd_ref[1], id_at(p + 1))

        f1.wait_recv()
        b1.wait_recv()
        c_rel_even.wait_recv()

        @pl.when(is_even)
        def _():
            f2_even.start()
            b2_even.start()

        @pl.when(jnp.logical_not(is_even))
        def _():
            f2_odd.start()
            b2_odd.start()

        gemm_silu_store(fwd_ref[2], id_at(p - 2))
        gemm_silu_store(bwd_ref[2], id_at(p + 2))
        gemm_silu_store(ant_ref[...], id_at(p + 4))

        far = id_at(jnp.where(is_even, p - 3, p + 3))
        f2_odd.wait_recv()
        gemm_silu_store(ftop_ref[...], far, row0=0)
        b2_odd.wait_recv()
        gemm_silu_store(fbot_ref[...], far, row0=half)

        for d in (f0, f1, f2_odd, b0, b1, b2_odd, c_own, c_rel_even):
            d.wait_send()

    out_shape = jax.ShapeDtypeStruct((N_DEV * m_per, n_per), jnp.float32)
    return pl.pallas_call(
        body,
        out_shape=out_shape,
        in_specs=[
            pl.BlockSpec(memory_space=pltpu.VMEM),
            pl.BlockSpec(memory_space=pltpu.VMEM),
        ],
        out_specs=pl.BlockSpec(memory_space=pltpu.VMEM),
        scratch_shapes=[
            pltpu.VMEM((3, m_per, k), jnp.float32),
            pltpu.VMEM((3, m_per, k), jnp.float32),
            pltpu.VMEM((m_per, k), jnp.float32),
            pltpu.VMEM((m_per, k), jnp.float32),
            pltpu.VMEM((half, k), jnp.float32),
            pltpu.VMEM((half, k), jnp.float32),
            pltpu.SemaphoreType.DMA((3,)),
            pltpu.SemaphoreType.DMA((3,)),
            pltpu.SemaphoreType.DMA((3,)),
            pltpu.SemaphoreType.DMA((3,)),
            pltpu.SemaphoreType.DMA((2,)),
            pltpu.SemaphoreType.DMA((2,)),
        ],
        compiler_params=pltpu.CompilerParams(collective_id=0),
    )(x, w_mat)
